# baseline (speedup 1.0000x reference)
import functools

import jax
import jax.numpy as jnp
from jax import lax
from jax.experimental import pallas as pl
from jax.experimental.pallas import tpu as pltpu
from jax.experimental.pallas import tpu_sc as plsc

N = 10000
E = 320000
D = 128
G = 64
GSZ = 5

NC = 2
NS = 16
NW = NC * NS
CHUNK = 128
NCHUNKS = E // CHUNK
RB = 200
NB = N // RB


def _sc_agg_body(table_hbm, src_hbm, dst_hbm, out_hbm,
                 src0_v, src1_v, src2_v, dst0_v, dst1_v, dst2_v,
                 rows0_v, rows1_v, rows2_v, acc_sh,
                 gsem0, gsem1, gsem2):
    c = lax.axis_index("c")
    s = lax.axis_index("s")
    w = s * NC + c

    zero16 = jnp.zeros((16,), jnp.float32)

    def _zero_body(i, carry):
        for j in range(D // 16):
            rows0_v[i, pl.ds(j * 16, 16)] = zero16
        return carry

    lax.fori_loop(0, CHUNK, _zero_body, 0)

    nzb = N // CHUNK

    def _zinit(k, carry):
        bid = s + NS * k

        @pl.when(bid < nzb)
        def _():
            off = pl.multiple_of(bid * CHUNK, 8)
            pltpu.sync_copy(rows0_v, acc_sh.at[pl.ds(off, CHUNK)])

        return carry

    lax.fori_loop(0, (nzb + NS - 1) // NS, _zinit, 0)

    @pl.when(s == 0)
    def _():
        pltpu.sync_copy(rows0_v.at[pl.ds(0, 16)],
                        acc_sh.at[pl.ds(nzb * CHUNK, 16)])

    plsc.subcore_barrier()

    def _ldidx(cid, sbuf, dbuf):
        off = pl.multiple_of(cid * CHUNK, 8)
        pltpu.sync_copy(src_hbm.at[pl.ds(off, CHUNK)], sbuf)
        pltpu.sync_copy(dst_hbm.at[pl.ds(off, CHUNK)], dbuf)

    def _gather(sbuf, buf, sem):
        return pltpu.async_copy(table_hbm.at[sbuf], buf, sem)

    def _gwait(sbuf, buf, sem):
        pltpu.make_async_copy(table_hbm.at[sbuf], buf, sem).wait()

    def _scat(dbuf, buf):
        pltpu.sync_copy(buf, acc_sh.at[dbuf], add=True)

    bufs = ((src0_v, dst0_v, rows0_v, gsem0),
            (src1_v, dst1_v, rows1_v, gsem1),
            (src2_v, dst2_v, rows2_v, gsem2))
    NSLOT = NCHUNKS // NW

    for u in range(3):
        sb, db, rb, gs = bufs[u]
        _ldidx(w + u * NW, sb, db)
        _gather(sb, rb, gs)

    def _edge_body(i, carry):
        for u in range(3):
            sb, db, rb, gs = bufs[u]
            slot = 3 * i + u
            _gwait(sb, rb, gs)
            _scat(db, rb)

            @pl.when(slot + 3 < NSLOT)
            def _():
                _ldidx(w + (slot + 3) * NW, sb, db)
                _gather(sb, rb, gs)

        return carry

    lax.fori_loop(0, NSLOT // 3, _edge_body, 0)

    @pl.when(w + NSLOT * NW < NCHUNKS)
    def _():
        _ldidx(w + NSLOT * NW, src0_v, dst0_v)
        _gather(src0_v, rows0_v, gsem0).wait()
        _scat(dst0_v, rows0_v)

    plsc.subcore_barrier()

    def _dump(k, carry):
        bid = s + NS * k

        @pl.when(bid < NB)
        def _():
            off = pl.multiple_of(bid * RB, 8)
            pltpu.sync_copy(acc_sh.at[pl.ds(off, RB)],
                            out_hbm.at[c, pl.ds(off, RB)])

        return carry

    lax.fori_loop(0, (NB + NS - 1) // NS, _dump, 0)


def _sc_pass(table, src, dst):
    mesh = plsc.VectorSubcoreMesh(core_axis_name="c", subcore_axis_name="s")
    kern = pl.kernel(
        _sc_agg_body,
        mesh=mesh,
        out_type=jax.ShapeDtypeStruct((NC, N, D), jnp.float32),
        scratch_types=[
            pltpu.VMEM((CHUNK,), jnp.int32),
            pltpu.VMEM((CHUNK,), jnp.int32),
            pltpu.VMEM((CHUNK,), jnp.int32),
            pltpu.VMEM((CHUNK,), jnp.int32),
            pltpu.VMEM((CHUNK,), jnp.int32),
            pltpu.VMEM((CHUNK,), jnp.int32),
            pltpu.VMEM((CHUNK, D), jnp.float32),
            pltpu.VMEM((CHUNK, D), jnp.float32),
            pltpu.VMEM((CHUNK, D), jnp.float32),
            pltpu.VMEM_SHARED((N, D), jnp.float32),
            pltpu.SemaphoreType.DMA,
            pltpu.SemaphoreType.DMA,
            pltpu.SemaphoreType.DMA,
        ],
    )
    return kern(table, src, dst)


ROWS_BLK = 1000
GRID = N // ROWS_BLK


def _mlp1_body(x_ref, pa_ref, w1_ref, b1_ref, w2_ref, b2_ref, out_ref):
    h = x_ref[...] + pa_ref[0] + pa_ref[1]
    t = jnp.maximum(
        jnp.dot(h, w1_ref[...], preferred_element_type=jnp.float32)
        + b1_ref[...], 0.0)
    o = (jnp.dot(t, w2_ref[...], preferred_element_type=jnp.float32)
         + b2_ref[...])
    out_ref[...] = jnp.maximum(o, 0.0)


def _mlp1(x, pa, w1, b1, w2, b2):
    return pl.pallas_call(
        _mlp1_body,
        grid=(GRID,),
        in_specs=[
            pl.BlockSpec((ROWS_BLK, D), lambda i: (i, 0)),
            pl.BlockSpec((NC, ROWS_BLK, D), lambda i: (0, i, 0)),
            pl.BlockSpec((D, D), lambda i: (0, 0)),
            pl.BlockSpec((1, D), lambda i: (0, 0)),
            pl.BlockSpec((D, D), lambda i: (0, 0)),
            pl.BlockSpec((1, D), lambda i: (0, 0)),
        ],
        out_specs=pl.BlockSpec((ROWS_BLK, D), lambda i: (i, 0)),
        out_shape=jax.ShapeDtypeStruct((N, D), jnp.float32),
    )(x, pa, w1, b1, w2, b2)


def _mlp2_body(h_ref, pa_ref, w1_ref, b1_ref, w2_ref, b2_ref,
               e_ref, g_ref, out_ref):
    i = pl.program_id(0)
    hin = h_ref[...] + pa_ref[0] + pa_ref[1]
    t = jnp.maximum(
        jnp.dot(hin, w1_ref[...], preferred_element_type=jnp.float32)
        + b1_ref[...], 0.0)
    h2 = (jnp.dot(t, w2_ref[...], preferred_element_type=jnp.float32)
          + b2_ref[...])
    contrib = lax.dot_general(e_ref[...], h2, (((0,), (0,)), ((), ())),
                              preferred_element_type=jnp.float32)

    @pl.when(i == 0)
    def _():
        out_ref[...] = jnp.zeros_like(out_ref)

    out_ref[...] += contrib

    @pl.when(i == pl.num_programs(0) - 1)
    def _():
        den = jnp.sum(g_ref[...], axis=1, keepdims=True) + 1e-4
        out_ref[...] = out_ref[...] / den


def _mlp2(h, pa, w1, b1, w2, b2, e_rep, ghg):
    return pl.pallas_call(
        _mlp2_body,
        grid=(GRID,),
        in_specs=[
            pl.BlockSpec((ROWS_BLK, D), lambda i: (i, 0)),
            pl.BlockSpec((NC, ROWS_BLK, D), lambda i: (0, i, 0)),
            pl.BlockSpec((D, D), lambda i: (0, 0)),
            pl.BlockSpec((1, D), lambda i: (0, 0)),
            pl.BlockSpec((D, D), lambda i: (0, 0)),
            pl.BlockSpec((1, D), lambda i: (0, 0)),
            pl.BlockSpec((ROWS_BLK, G), lambda i: (i, 0)),
            pl.BlockSpec((G, N // GSZ), lambda i: (0, 0)),
        ],
        out_specs=pl.BlockSpec((G, D), lambda i: (0, 0)),
        out_shape=jax.ShapeDtypeStruct((G, D), jnp.float32),
    )(h, pa, w1, b1, w2, b2, e_rep, ghg)


def kernel(x, edge_index, graph_has_graphlet,
           W1a, b1a, W2a, b2a, W1b, b1b, W2b, b2b):
    src = edge_index[0]
    dst = edge_index[1]
    b1a2 = b1a.reshape(1, D)
    b2a2 = b2a.reshape(1, D)
    b1b2 = b1b.reshape(1, D)
    b2b2 = b2b.reshape(1, D)
    ghg_rep_t = jnp.repeat(graph_has_graphlet.T, GSZ, axis=0)

    pa1 = _sc_pass(x, src, dst)
    h1r = _mlp1(x, pa1, W1a, b1a2, W2a, b2a2)
    pa2 = _sc_pass(h1r, src, dst)
    out = _mlp2(h1r, pa2, W1b, b1b2, W2b, b2b2, ghg_rep_t, graph_has_graphlet)
    return out


# speedup vs baseline: 1.0222x; 1.0222x over previous
import functools

import jax
import jax.numpy as jnp
from jax import lax
from jax.experimental import pallas as pl
from jax.experimental.pallas import tpu as pltpu
from jax.experimental.pallas import tpu_sc as plsc

N = 10000
E = 320000
D = 128
G = 64
GSZ = 5

NC = 2
NS = 16
NW = NC * NS
CHUNK = 128
NCHUNKS = E // CHUNK
RB = 200
NB = N // RB


def _sc_agg_body(table_hbm, edge_hbm, out_hbm,
                 src0_v, src1_v, src2_v, dst0_v, dst1_v, dst2_v,
                 rows0_v, rows1_v, rows2_v, acc_sh,
                 gsem0, gsem1, gsem2):
    c = lax.axis_index("c")
    s = lax.axis_index("s")
    w = s * NC + c

    zero16 = jnp.zeros((16,), jnp.float32)

    def _zero_body(i, carry):
        for j in range(D // 16):
            rows0_v[i, pl.ds(j * 16, 16)] = zero16
        return carry

    lax.fori_loop(0, CHUNK, _zero_body, 0)

    nzb = N // CHUNK

    def _zinit(k, carry):
        bid = s + NS * k

        @pl.when(bid < nzb)
        def _():
            off = pl.multiple_of(bid * CHUNK, 8)

            @pl.when(c == 0)
            def _():
                pltpu.sync_copy(table_hbm.at[pl.ds(off, CHUNK)],
                                acc_sh.at[pl.ds(off, CHUNK)])

            @pl.when(c == 1)
            def _():
                pltpu.sync_copy(rows0_v, acc_sh.at[pl.ds(off, CHUNK)])

        return carry

    lax.fori_loop(0, (nzb + NS - 1) // NS, _zinit, 0)

    @pl.when(s == 0)
    def _():
        toff = pl.multiple_of(nzb * CHUNK, 8)

        @pl.when(c == 0)
        def _():
            pltpu.sync_copy(table_hbm.at[pl.ds(toff, 16)],
                            acc_sh.at[pl.ds(toff, 16)])

        @pl.when(c == 1)
        def _():
            pltpu.sync_copy(rows0_v.at[pl.ds(0, 16)],
                            acc_sh.at[pl.ds(toff, 16)])

    plsc.subcore_barrier()

    def _ldidx(cid, sbuf, dbuf):
        off = pl.multiple_of(cid * CHUNK, 8)
        pltpu.sync_copy(edge_hbm.at[0, pl.ds(off, CHUNK)], sbuf)
        pltpu.sync_copy(edge_hbm.at[1, pl.ds(off, CHUNK)], dbuf)

    def _gather(sbuf, buf, sem):
        return pltpu.async_copy(table_hbm.at[sbuf], buf, sem)

    def _gwait(sbuf, buf, sem):
        pltpu.make_async_copy(table_hbm.at[sbuf], buf, sem).wait()

    def _scat(dbuf, buf):
        pltpu.sync_copy(buf, acc_sh.at[dbuf], add=True)

    bufs = ((src0_v, dst0_v, rows0_v, gsem0),
            (src1_v, dst1_v, rows1_v, gsem1),
            (src2_v, dst2_v, rows2_v, gsem2))
    NSLOT = NCHUNKS // NW

    for u in range(3):
        sb, db, rb, gs = bufs[u]
        _ldidx(w + u * NW, sb, db)
        _gather(sb, rb, gs)

    def _edge_body(i, carry):
        for u in range(3):
            sb, db, rb, gs = bufs[u]
            slot = 3 * i + u
            _gwait(sb, rb, gs)
            _scat(db, rb)

            @pl.when(slot + 3 < NSLOT)
            def _():
                _ldidx(w + (slot + 3) * NW, sb, db)
                _gather(sb, rb, gs)

        return carry

    lax.fori_loop(0, NSLOT // 3, _edge_body, 0)

    @pl.when(w + NSLOT * NW < NCHUNKS)
    def _():
        _ldidx(w + NSLOT * NW, src0_v, dst0_v)
        _gather(src0_v, rows0_v, gsem0).wait()
        _scat(dst0_v, rows0_v)

    plsc.subcore_barrier()

    def _dump(k, carry):
        bid = s + NS * k

        @pl.when(bid < NB)
        def _():
            off = pl.multiple_of(bid * RB, 8)
            pltpu.sync_copy(acc_sh.at[pl.ds(off, RB)],
                            out_hbm.at[c, pl.ds(off, RB)])

        return carry

    lax.fori_loop(0, (NB + NS - 1) // NS, _dump, 0)


def _sc_pass(table, edge_index):
    mesh = plsc.VectorSubcoreMesh(core_axis_name="c", subcore_axis_name="s")
    kern = pl.kernel(
        _sc_agg_body,
        mesh=mesh,
        out_type=jax.ShapeDtypeStruct((NC, N, D), jnp.float32),
        scratch_types=[
            pltpu.VMEM((CHUNK,), jnp.int32),
            pltpu.VMEM((CHUNK,), jnp.int32),
            pltpu.VMEM((CHUNK,), jnp.int32),
            pltpu.VMEM((CHUNK,), jnp.int32),
            pltpu.VMEM((CHUNK,), jnp.int32),
            pltpu.VMEM((CHUNK,), jnp.int32),
            pltpu.VMEM((CHUNK, D), jnp.float32),
            pltpu.VMEM((CHUNK, D), jnp.float32),
            pltpu.VMEM((CHUNK, D), jnp.float32),
            pltpu.VMEM_SHARED((N, D), jnp.float32),
            pltpu.SemaphoreType.DMA,
            pltpu.SemaphoreType.DMA,
            pltpu.SemaphoreType.DMA,
        ],
    )
    return kern(table, edge_index)


ROWS_BLK = 1000
GRID = N // ROWS_BLK


def _mlp1_body(pa_ref, w1_ref, b1_ref, w2_ref, b2_ref, out_ref):
    h = pa_ref[0] + pa_ref[1]
    t = jnp.maximum(
        jnp.dot(h, w1_ref[...], preferred_element_type=jnp.float32)
        + b1_ref[...], 0.0)
    o = (jnp.dot(t, w2_ref[...], preferred_element_type=jnp.float32)
         + b2_ref[...])
    out_ref[...] = jnp.maximum(o, 0.0)


def _mlp1(pa, w1, b1, w2, b2):
    return pl.pallas_call(
        _mlp1_body,
        grid=(GRID,),
        in_specs=[
            pl.BlockSpec((NC, ROWS_BLK, D), lambda i: (0, i, 0)),
            pl.BlockSpec((D, D), lambda i: (0, 0)),
            pl.BlockSpec((1, D), lambda i: (0, 0)),
            pl.BlockSpec((D, D), lambda i: (0, 0)),
            pl.BlockSpec((1, D), lambda i: (0, 0)),
        ],
        out_specs=pl.BlockSpec((ROWS_BLK, D), lambda i: (i, 0)),
        out_shape=jax.ShapeDtypeStruct((N, D), jnp.float32),
    )(pa, w1, b1, w2, b2)


def _mlp2_body(pa_ref, w1_ref, b1_ref, w2_ref, b2_ref,
               e_ref, g_ref, out_ref):
    i = pl.program_id(0)
    hin = pa_ref[0] + pa_ref[1]
    t = jnp.maximum(
        jnp.dot(hin, w1_ref[...], preferred_element_type=jnp.float32)
        + b1_ref[...], 0.0)
    h2 = (jnp.dot(t, w2_ref[...], preferred_element_type=jnp.float32)
          + b2_ref[...])
    contrib = lax.dot_general(e_ref[...], h2, (((0,), (0,)), ((), ())),
                              preferred_element_type=jnp.float32)

    @pl.when(i == 0)
    def _():
        out_ref[...] = jnp.zeros_like(out_ref)

    out_ref[...] += contrib

    @pl.when(i == pl.num_programs(0) - 1)
    def _():
        den = jnp.sum(g_ref[...], axis=1, keepdims=True) + 1e-4
        out_ref[...] = out_ref[...] / den


def _mlp2(pa, w1, b1, w2, b2, e_rep, ghg):
    return pl.pallas_call(
        _mlp2_body,
        grid=(GRID,),
        in_specs=[
            pl.BlockSpec((NC, ROWS_BLK, D), lambda i: (0, i, 0)),
            pl.BlockSpec((D, D), lambda i: (0, 0)),
            pl.BlockSpec((1, D), lambda i: (0, 0)),
            pl.BlockSpec((D, D), lambda i: (0, 0)),
            pl.BlockSpec((1, D), lambda i: (0, 0)),
            pl.BlockSpec((ROWS_BLK, G), lambda i: (i, 0)),
            pl.BlockSpec((G, N // GSZ), lambda i: (0, 0)),
        ],
        out_specs=pl.BlockSpec((G, D), lambda i: (0, 0)),
        out_shape=jax.ShapeDtypeStruct((G, D), jnp.float32),
    )(pa, w1, b1, w2, b2, e_rep, ghg)


def kernel(x, edge_index, graph_has_graphlet,
           W1a, b1a, W2a, b2a, W1b, b1b, W2b, b2b):
    b1a2 = b1a.reshape(1, D)
    b2a2 = b2a.reshape(1, D)
    b1b2 = b1b.reshape(1, D)
    b2b2 = b2b.reshape(1, D)
    ghg_rep_t = jnp.repeat(graph_has_graphlet.T, GSZ, axis=0)

    pa1 = _sc_pass(x, edge_index)
    h1r = _mlp1(pa1, W1a, b1a2, W2a, b2a2)
    pa2 = _sc_pass(h1r, edge_index)
    out = _mlp2(pa2, W1b, b1b2, W2b, b2b2, ghg_rep_t, graph_has_graphlet)
    return out


# speedup vs baseline: 1.3387x; 1.3097x over previous
import functools

import jax
import jax.numpy as jnp
from jax import lax
from jax.experimental import pallas as pl
from jax.experimental.pallas import tpu as pltpu
from jax.experimental.pallas import tpu_sc as plsc

N = 10000
E = 320000
D = 128
G = 64
GSZ = 5

NC = 2
NS = 16
NW = NC * NS
CHUNK = 128
NCHUNKS = E // CHUNK
RB = 200
NB = N // RB


def _sc_agg_body(table_hbm, edge_hbm, out_hbm,
                 src0_v, src1_v, src2_v, src3_v, src4_v, src5_v,
                 dst0_v, dst1_v, dst2_v, dst3_v, dst4_v, dst5_v,
                 rows0_v, rows1_v, acc_sh,
                 gsem0, gsem1,
                 isem0, isem1, isem2, isem3, isem4, isem5):
    c = lax.axis_index("c")
    s = lax.axis_index("s")
    w = s * NC + c

    zero16 = jnp.zeros((16,), jnp.float32)

    def _zero_body(i, carry):
        for j in range(D // 16):
            rows0_v[i, pl.ds(j * 16, 16)] = zero16
        return carry

    lax.fori_loop(0, CHUNK, _zero_body, 0)

    nzb = N // CHUNK

    def _zinit(k, carry):
        bid = s + NS * k

        @pl.when(bid < nzb)
        def _():
            off = pl.multiple_of(bid * CHUNK, 8)

            @pl.when(c == 0)
            def _():
                pltpu.sync_copy(table_hbm.at[pl.ds(off, CHUNK)],
                                acc_sh.at[pl.ds(off, CHUNK)])

            @pl.when(c == 1)
            def _():
                pltpu.sync_copy(rows0_v, acc_sh.at[pl.ds(off, CHUNK)])

        return carry

    lax.fori_loop(0, (nzb + NS - 1) // NS, _zinit, 0)

    @pl.when(s == 0)
    def _():
        toff = pl.multiple_of(nzb * CHUNK, 8)

        @pl.when(c == 0)
        def _():
            pltpu.sync_copy(table_hbm.at[pl.ds(toff, 16)],
                            acc_sh.at[pl.ds(toff, 16)])

        @pl.when(c == 1)
        def _():
            pltpu.sync_copy(rows0_v.at[pl.ds(0, 16)],
                            acc_sh.at[pl.ds(toff, 16)])

    plsc.subcore_barrier()

    sbufs = (src0_v, src1_v, src2_v, src3_v, src4_v, src5_v)
    dbufs = (dst0_v, dst1_v, dst2_v, dst3_v, dst4_v, dst5_v)
    isems = (isem0, isem1, isem2, isem3, isem4, isem5)
    rbufs = (rows0_v, rows1_v)
    gsems = (gsem0, gsem1)
    NSLOT = NCHUNKS // NW

    def _ldidx(slot, u):
        off = pl.multiple_of((w + slot * NW) * CHUNK, 8)
        pltpu.async_copy(edge_hbm.at[0, pl.ds(off, CHUNK)], sbufs[u],
                         isems[u])
        pltpu.async_copy(edge_hbm.at[1, pl.ds(off, CHUNK)], dbufs[u],
                         isems[u])

    def _iwait(slot, u):
        off = pl.multiple_of((w + slot * NW) * CHUNK, 8)
        pltpu.make_async_copy(edge_hbm.at[0, pl.ds(off, CHUNK)], sbufs[u],
                              isems[u]).wait()
        pltpu.make_async_copy(edge_hbm.at[1, pl.ds(off, CHUNK)], dbufs[u],
                              isems[u]).wait()

    def _gather(u, r):
        pltpu.async_copy(table_hbm.at[sbufs[u]], rbufs[r], gsems[r])

    def _gwait(u, r):
        pltpu.make_async_copy(table_hbm.at[sbufs[u]], rbufs[r],
                              gsems[r]).wait()

    def _scat(u, r):
        pltpu.sync_copy(rbufs[r], acc_sh.at[dbufs[u]], add=True)

    for j in range(6):
        _ldidx(j, j)
    for j in range(2):
        _iwait(j, j)
        _gather(j, j)

    def _edge_body(i, carry):
        for u in range(6):
            slot = 6 * i + u
            r = u % 2
            _gwait(u, r)
            _scat(u, r)

            @pl.when(slot + 6 < NSLOT)
            def _():
                _ldidx(slot + 6, u)

            @pl.when(slot + 2 < NSLOT)
            def _():
                _iwait(slot + 2, (u + 2) % 6)
                _gather((u + 2) % 6, r)

        return carry

    lax.fori_loop(0, NSLOT // 6, _edge_body, 0)

    @pl.when(w + NSLOT * NW < NCHUNKS)
    def _():
        off = pl.multiple_of((w + NSLOT * NW) * CHUNK, 8)
        pltpu.sync_copy(edge_hbm.at[0, pl.ds(off, CHUNK)], src0_v)
        pltpu.sync_copy(edge_hbm.at[1, pl.ds(off, CHUNK)], dst0_v)
        pltpu.async_copy(table_hbm.at[src0_v], rows0_v, gsem0).wait()
        pltpu.sync_copy(rows0_v, acc_sh.at[dst0_v], add=True)

    plsc.subcore_barrier()

    def _dump(k, carry):
        bid = s + NS * k

        @pl.when(bid < NB)
        def _():
            off = pl.multiple_of(bid * RB, 8)
            pltpu.sync_copy(acc_sh.at[pl.ds(off, RB)],
                            out_hbm.at[c, pl.ds(off, RB)])

        return carry

    lax.fori_loop(0, (NB + NS - 1) // NS, _dump, 0)


def _sc_pass(table, edge_index):
    mesh = plsc.VectorSubcoreMesh(core_axis_name="c", subcore_axis_name="s")
    kern = pl.kernel(
        _sc_agg_body,
        mesh=mesh,
        out_type=jax.ShapeDtypeStruct((NC, N, D), jnp.float32),
        scratch_types=(
            [pltpu.VMEM((CHUNK,), jnp.int32)] * 12
            + [pltpu.VMEM((CHUNK, D), jnp.float32)] * 2
            + [pltpu.VMEM_SHARED((N, D), jnp.float32)]
            + [pltpu.SemaphoreType.DMA] * 8
        ),
    )
    return kern(table, edge_index)


ROWS_BLK = 1000
GRID = N // ROWS_BLK


def _mlp1_body(pa_ref, w1_ref, b1_ref, w2_ref, b2_ref, out_ref):
    h = pa_ref[0] + pa_ref[1]
    t = jnp.maximum(
        jnp.dot(h, w1_ref[...], preferred_element_type=jnp.float32)
        + b1_ref[...], 0.0)
    o = (jnp.dot(t, w2_ref[...], preferred_element_type=jnp.float32)
         + b2_ref[...])
    out_ref[...] = jnp.maximum(o, 0.0)


def _mlp1(pa, w1, b1, w2, b2):
    return pl.pallas_call(
        _mlp1_body,
        grid=(GRID,),
        in_specs=[
            pl.BlockSpec((NC, ROWS_BLK, D), lambda i: (0, i, 0)),
            pl.BlockSpec((D, D), lambda i: (0, 0)),
            pl.BlockSpec((1, D), lambda i: (0, 0)),
            pl.BlockSpec((D, D), lambda i: (0, 0)),
            pl.BlockSpec((1, D), lambda i: (0, 0)),
        ],
        out_specs=pl.BlockSpec((ROWS_BLK, D), lambda i: (i, 0)),
        out_shape=jax.ShapeDtypeStruct((N, D), jnp.float32),
    )(pa, w1, b1, w2, b2)


def _mlp2_body(pa_ref, w1_ref, b1_ref, w2_ref, b2_ref,
               e_ref, g_ref, out_ref):
    i = pl.program_id(0)
    hin = pa_ref[0] + pa_ref[1]
    t = jnp.maximum(
        jnp.dot(hin, w1_ref[...], preferred_element_type=jnp.float32)
        + b1_ref[...], 0.0)
    h2 = (jnp.dot(t, w2_ref[...], preferred_element_type=jnp.float32)
          + b2_ref[...])
    contrib = lax.dot_general(e_ref[...], h2, (((0,), (0,)), ((), ())),
                              preferred_element_type=jnp.float32)

    @pl.when(i == 0)
    def _():
        out_ref[...] = jnp.zeros_like(out_ref)

    out_ref[...] += contrib

    @pl.when(i == pl.num_programs(0) - 1)
    def _():
        den = jnp.sum(g_ref[...], axis=1, keepdims=True) + 1e-4
        out_ref[...] = out_ref[...] / den


def _mlp2(pa, w1, b1, w2, b2, e_rep, ghg):
    return pl.pallas_call(
        _mlp2_body,
        grid=(GRID,),
        in_specs=[
            pl.BlockSpec((NC, ROWS_BLK, D), lambda i: (0, i, 0)),
            pl.BlockSpec((D, D), lambda i: (0, 0)),
            pl.BlockSpec((1, D), lambda i: (0, 0)),
            pl.BlockSpec((D, D), lambda i: (0, 0)),
            pl.BlockSpec((1, D), lambda i: (0, 0)),
            pl.BlockSpec((ROWS_BLK, G), lambda i: (i, 0)),
            pl.BlockSpec((G, N // GSZ), lambda i: (0, 0)),
        ],
        out_specs=pl.BlockSpec((G, D), lambda i: (0, 0)),
        out_shape=jax.ShapeDtypeStruct((G, D), jnp.float32),
    )(pa, w1, b1, w2, b2, e_rep, ghg)


def kernel(x, edge_index, graph_has_graphlet,
           W1a, b1a, W2a, b2a, W1b, b1b, W2b, b2b):
    b1a2 = b1a.reshape(1, D)
    b2a2 = b2a.reshape(1, D)
    b1b2 = b1b.reshape(1, D)
    b2b2 = b2b.reshape(1, D)
    ghg_rep_t = jnp.repeat(graph_has_graphlet.T, GSZ, axis=0)

    pa1 = _sc_pass(x, edge_index)
    h1r = _mlp1(pa1, W1a, b1a2, W2a, b2a2)
    pa2 = _sc_pass(h1r, edge_index)
    out = _mlp2(pa2, W1b, b1b2, W2b, b2b2, ghg_rep_t, graph_has_graphlet)
    return out


# speedup vs baseline: 1.4021x; 1.0474x over previous
import functools

import jax
import jax.numpy as jnp
from jax import lax
from jax.experimental import pallas as pl
from jax.experimental.pallas import tpu as pltpu
from jax.experimental.pallas import tpu_sc as plsc

N = 10000
E = 320000
D = 128
G = 64
GSZ = 5

NC = 2
NS = 16
NW = NC * NS
CHUNK = 128
NCHUNKS = E // CHUNK
RB = 200
NB = N // RB


def _sc_agg_body(table_hbm, edge_hbm, out_hbm,
                 src0_v, src1_v, src2_v, src3_v, src4_v, src5_v,
                 dst0_v, dst1_v, dst2_v, dst3_v, dst4_v, dst5_v,
                 rows0_v, rows1_v, rows2_v, acc_sh,
                 gsem0, gsem1, gsem2,
                 isem0, isem1, isem2, isem3, isem4, isem5,
                 ssem0, ssem1, ssem2):
    c = lax.axis_index("c")
    s = lax.axis_index("s")
    w = s * NC + c

    zero16 = jnp.zeros((16,), jnp.float32)

    def _zero_body(i, carry):
        for j in range(D // 16):
            rows0_v[i, pl.ds(j * 16, 16)] = zero16
        return carry

    lax.fori_loop(0, CHUNK, _zero_body, 0)

    nzb = N // CHUNK

    def _zinit(k, carry):
        bid = s + NS * k

        @pl.when(bid < nzb)
        def _():
            off = pl.multiple_of(bid * CHUNK, 8)

            @pl.when(c == 0)
            def _():
                pltpu.sync_copy(table_hbm.at[pl.ds(off, CHUNK)],
                                acc_sh.at[pl.ds(off, CHUNK)])

            @pl.when(c == 1)
            def _():
                pltpu.sync_copy(rows0_v, acc_sh.at[pl.ds(off, CHUNK)])

        return carry

    lax.fori_loop(0, (nzb + NS - 1) // NS, _zinit, 0)

    @pl.when(s == 0)
    def _():
        toff = pl.multiple_of(nzb * CHUNK, 8)

        @pl.when(c == 0)
        def _():
            pltpu.sync_copy(table_hbm.at[pl.ds(toff, 16)],
                            acc_sh.at[pl.ds(toff, 16)])

        @pl.when(c == 1)
        def _():
            pltpu.sync_copy(rows0_v.at[pl.ds(0, 16)],
                            acc_sh.at[pl.ds(toff, 16)])

    plsc.subcore_barrier()

    sbufs = (src0_v, src1_v, src2_v, src3_v, src4_v, src5_v)
    dbufs = (dst0_v, dst1_v, dst2_v, dst3_v, dst4_v, dst5_v)
    isems = (isem0, isem1, isem2, isem3, isem4, isem5)
    rbufs = (rows0_v, rows1_v, rows2_v)
    gsems = (gsem0, gsem1, gsem2)
    ssems = (ssem0, ssem1, ssem2)
    NSLOT = NCHUNKS // NW

    def _ldidx(slot, u):
        off = pl.multiple_of((w + slot * NW) * CHUNK, 8)
        pltpu.async_copy(edge_hbm.at[0, pl.ds(off, CHUNK)], sbufs[u],
                         isems[u])
        pltpu.async_copy(edge_hbm.at[1, pl.ds(off, CHUNK)], dbufs[u],
                         isems[u])

    def _iwait(slot, u):
        off = pl.multiple_of((w + slot * NW) * CHUNK, 8)
        pltpu.make_async_copy(edge_hbm.at[0, pl.ds(off, CHUNK)], sbufs[u],
                              isems[u]).wait()
        pltpu.make_async_copy(edge_hbm.at[1, pl.ds(off, CHUNK)], dbufs[u],
                              isems[u]).wait()

    def _gather(u, r):
        pltpu.async_copy(table_hbm.at[sbufs[u]], rbufs[r], gsems[r])

    def _gwait(u, r):
        pltpu.make_async_copy(table_hbm.at[sbufs[u]], rbufs[r],
                              gsems[r]).wait()

    def _scat(u, r):
        pltpu.async_copy(rbufs[r], acc_sh.at[dbufs[u]], ssems[r], add=True)

    def _swait(u, r):
        pltpu.make_async_copy(rbufs[r], acc_sh.at[dbufs[u]],
                              ssems[r]).wait()

    for j in range(4):
        _ldidx(j, j)
    for j in range(2):
        _iwait(j, j)
        _gather(j, j)

    def _edge_body(i, carry):
        for u in range(6):
            slot = 6 * i + u
            r = u % 3
            _gwait(u, r)
            _scat(u, r)

            @pl.when(slot + 4 < NSLOT)
            def _():
                _ldidx(slot + 4, (u + 4) % 6)

            @pl.when(slot + 2 < NSLOT)
            def _():
                _iwait(slot + 2, (u + 2) % 6)

                @pl.when(slot >= 1)
                def _():
                    _swait((u + 5) % 6, (r + 2) % 3)

                _gather((u + 2) % 6, (r + 2) % 3)

        return carry

    lax.fori_loop(0, NSLOT // 6, _edge_body, 0)
    for t in (NSLOT - 3, NSLOT - 2, NSLOT - 1):
        _swait(t % 6, t % 3)

    @pl.when(w + NSLOT * NW < NCHUNKS)
    def _():
        off = pl.multiple_of((w + NSLOT * NW) * CHUNK, 8)
        pltpu.sync_copy(edge_hbm.at[0, pl.ds(off, CHUNK)], src0_v)
        pltpu.sync_copy(edge_hbm.at[1, pl.ds(off, CHUNK)], dst0_v)
        pltpu.async_copy(table_hbm.at[src0_v], rows0_v, gsem0).wait()
        pltpu.sync_copy(rows0_v, acc_sh.at[dst0_v], add=True)

    plsc.subcore_barrier()

    def _dump(k, carry):
        bid = s + NS * k

        @pl.when(bid < NB)
        def _():
            off = pl.multiple_of(bid * RB, 8)
            pltpu.sync_copy(acc_sh.at[pl.ds(off, RB)],
                            out_hbm.at[c, pl.ds(off, RB)])

        return carry

    lax.fori_loop(0, (NB + NS - 1) // NS, _dump, 0)


def _sc_pass(table, edge_index):
    mesh = plsc.VectorSubcoreMesh(core_axis_name="c", subcore_axis_name="s")
    kern = pl.kernel(
        _sc_agg_body,
        mesh=mesh,
        out_type=jax.ShapeDtypeStruct((NC, N, D), jnp.float32),
        scratch_types=(
            [pltpu.VMEM((CHUNK,), jnp.int32)] * 12
            + [pltpu.VMEM((CHUNK, D), jnp.float32)] * 3
            + [pltpu.VMEM_SHARED((N, D), jnp.float32)]
            + [pltpu.SemaphoreType.DMA] * 12
        ),
    )
    return kern(table, edge_index)


ROWS_BLK = 1000
GRID = N // ROWS_BLK


def _mlp1_body(pa_ref, w1_ref, b1_ref, w2_ref, b2_ref, out_ref):
    h = pa_ref[0] + pa_ref[1]
    t = jnp.maximum(
        jnp.dot(h, w1_ref[...], preferred_element_type=jnp.float32)
        + b1_ref[...], 0.0)
    o = (jnp.dot(t, w2_ref[...], preferred_element_type=jnp.float32)
         + b2_ref[...])
    out_ref[...] = jnp.maximum(o, 0.0)


def _mlp1(pa, w1, b1, w2, b2):
    return pl.pallas_call(
        _mlp1_body,
        grid=(GRID,),
        in_specs=[
            pl.BlockSpec((NC, ROWS_BLK, D), lambda i: (0, i, 0)),
            pl.BlockSpec((D, D), lambda i: (0, 0)),
            pl.BlockSpec((1, D), lambda i: (0, 0)),
            pl.BlockSpec((D, D), lambda i: (0, 0)),
            pl.BlockSpec((1, D), lambda i: (0, 0)),
        ],
        out_specs=pl.BlockSpec((ROWS_BLK, D), lambda i: (i, 0)),
        out_shape=jax.ShapeDtypeStruct((N, D), jnp.float32),
    )(pa, w1, b1, w2, b2)


def _mlp2_body(pa_ref, w1_ref, b1_ref, w2_ref, b2_ref,
               e_ref, g_ref, out_ref):
    i = pl.program_id(0)
    hin = pa_ref[0] + pa_ref[1]
    t = jnp.maximum(
        jnp.dot(hin, w1_ref[...], preferred_element_type=jnp.float32)
        + b1_ref[...], 0.0)
    h2 = (jnp.dot(t, w2_ref[...], preferred_element_type=jnp.float32)
          + b2_ref[...])
    contrib = lax.dot_general(e_ref[...], h2, (((0,), (0,)), ((), ())),
                              preferred_element_type=jnp.float32)

    @pl.when(i == 0)
    def _():
        out_ref[...] = jnp.zeros_like(out_ref)

    out_ref[...] += contrib

    @pl.when(i == pl.num_programs(0) - 1)
    def _():
        den = jnp.sum(g_ref[...], axis=1, keepdims=True) + 1e-4
        out_ref[...] = out_ref[...] / den


def _mlp2(pa, w1, b1, w2, b2, e_rep, ghg):
    return pl.pallas_call(
        _mlp2_body,
        grid=(GRID,),
        in_specs=[
            pl.BlockSpec((NC, ROWS_BLK, D), lambda i: (0, i, 0)),
            pl.BlockSpec((D, D), lambda i: (0, 0)),
            pl.BlockSpec((1, D), lambda i: (0, 0)),
            pl.BlockSpec((D, D), lambda i: (0, 0)),
            pl.BlockSpec((1, D), lambda i: (0, 0)),
            pl.BlockSpec((ROWS_BLK, G), lambda i: (i, 0)),
            pl.BlockSpec((G, N // GSZ), lambda i: (0, 0)),
        ],
        out_specs=pl.BlockSpec((G, D), lambda i: (0, 0)),
        out_shape=jax.ShapeDtypeStruct((G, D), jnp.float32),
    )(pa, w1, b1, w2, b2, e_rep, ghg)


def kernel(x, edge_index, graph_has_graphlet,
           W1a, b1a, W2a, b2a, W1b, b1b, W2b, b2b):
    b1a2 = b1a.reshape(1, D)
    b2a2 = b2a.reshape(1, D)
    b1b2 = b1b.reshape(1, D)
    b2b2 = b2b.reshape(1, D)
    ghg_rep_t = jnp.repeat(graph_has_graphlet.T, GSZ, axis=0)

    pa1 = _sc_pass(x, edge_index)
    h1r = _mlp1(pa1, W1a, b1a2, W2a, b2a2)
    pa2 = _sc_pass(h1r, edge_index)
    out = _mlp2(pa2, W1b, b1b2, W2b, b2b2, ghg_rep_t, graph_has_graphlet)
    return out


# speedup vs baseline: 1.4061x; 1.0028x over previous
import jax
import jax.numpy as jnp
from jax import lax
from jax.experimental import pallas as pl
from jax.experimental.pallas import tpu as pltpu
from jax.experimental.pallas import tpu_sc as plsc

N = 10000
E = 320000
D = 128
G = 64
GSZ = 5

NC = 2
NS = 16
NW = NC * NS
CHUNK = 128
NCHUNKS = E // CHUNK
RB = 200
NB = N // RB


def _sc_agg_body(table_hbm, edge_hbm, out_hbm,
                 src0_v, src1_v, src2_v, src3_v, src4_v, src5_v,
                 dst0_v, dst1_v, dst2_v, dst3_v, dst4_v, dst5_v,
                 rows0_v, rows1_v, rows2_v, acc_sh,
                 gsem0, gsem1, gsem2,
                 isem0, isem1, isem2, isem3, isem4, isem5,
                 ssem0, ssem1, ssem2):
    c = lax.axis_index("c")
    s = lax.axis_index("s")
    w = s * NC + c

    zero16 = jnp.zeros((16,), jnp.float32)

    def _zero_body(i, carry):
        for j in range(D // 16):
            rows0_v[i, pl.ds(j * 16, 16)] = zero16
        return carry

    lax.fori_loop(0, CHUNK, _zero_body, 0)

    nzb = N // CHUNK

    def _zinit(k, carry):
        bid = s + NS * k

        @pl.when(bid < nzb)
        def _():
            off = pl.multiple_of(bid * CHUNK, 8)

            @pl.when(c == 0)
            def _():
                pltpu.sync_copy(table_hbm.at[pl.ds(off, CHUNK)],
                                acc_sh.at[pl.ds(off, CHUNK)])

            @pl.when(c == 1)
            def _():
                pltpu.sync_copy(rows0_v, acc_sh.at[pl.ds(off, CHUNK)])

        return carry

    lax.fori_loop(0, (nzb + NS - 1) // NS, _zinit, 0)

    @pl.when(s == 0)
    def _():
        toff = pl.multiple_of(nzb * CHUNK, 8)

        @pl.when(c == 0)
        def _():
            pltpu.sync_copy(table_hbm.at[pl.ds(toff, 16)],
                            acc_sh.at[pl.ds(toff, 16)])

        @pl.when(c == 1)
        def _():
            pltpu.sync_copy(rows0_v.at[pl.ds(0, 16)],
                            acc_sh.at[pl.ds(toff, 16)])

    plsc.subcore_barrier()

    sbufs = (src0_v, src1_v, src2_v, src3_v, src4_v, src5_v)
    dbufs = (dst0_v, dst1_v, dst2_v, dst3_v, dst4_v, dst5_v)
    isems = (isem0, isem1, isem2, isem3, isem4, isem5)
    rbufs = (rows0_v, rows1_v, rows2_v)
    gsems = (gsem0, gsem1, gsem2)
    ssems = (ssem0, ssem1, ssem2)
    NSLOT = NCHUNKS // NW

    def _ldidx(slot, u):
        off = pl.multiple_of((w + slot * NW) * CHUNK, 8)
        pltpu.async_copy(edge_hbm.at[0, pl.ds(off, CHUNK)], sbufs[u],
                         isems[u])
        pltpu.async_copy(edge_hbm.at[1, pl.ds(off, CHUNK)], dbufs[u],
                         isems[u])

    def _iwait(slot, u):
        off = pl.multiple_of((w + slot * NW) * CHUNK, 8)
        pltpu.make_async_copy(edge_hbm.at[0, pl.ds(off, CHUNK)], sbufs[u],
                              isems[u]).wait()
        pltpu.make_async_copy(edge_hbm.at[1, pl.ds(off, CHUNK)], dbufs[u],
                              isems[u]).wait()

    def _gather(u, r):
        pltpu.async_copy(table_hbm.at[sbufs[u]], rbufs[r], gsems[r])

    def _gwait(u, r):
        pltpu.make_async_copy(table_hbm.at[sbufs[u]], rbufs[r],
                              gsems[r]).wait()

    def _scat(u, r):
        pltpu.async_copy(rbufs[r], acc_sh.at[dbufs[u]], ssems[r], add=True)

    def _swait(u, r):
        pltpu.make_async_copy(rbufs[r], acc_sh.at[dbufs[u]],
                              ssems[r]).wait()

    for j in range(4):
        _ldidx(j, j)
    for j in range(2):
        _iwait(j, j)
        _gather(j, j)

    def _edge_body(i, carry):
        for u in range(6):
            slot = 6 * i + u
            r = u % 3
            _gwait(u, r)
            _scat(u, r)

            @pl.when(slot + 4 < NSLOT)
            def _():
                _ldidx(slot + 4, (u + 4) % 6)

            @pl.when(slot + 2 < NSLOT)
            def _():
                _iwait(slot + 2, (u + 2) % 6)

                @pl.when(slot >= 1)
                def _():
                    _swait((u + 5) % 6, (r + 2) % 3)

                _gather((u + 2) % 6, (r + 2) % 3)

        return carry

    lax.fori_loop(0, NSLOT // 6, _edge_body, 0)
    for t in (NSLOT - 3, NSLOT - 2, NSLOT - 1):
        _swait(t % 6, t % 3)

    @pl.when(w + NSLOT * NW < NCHUNKS)
    def _():
        off = pl.multiple_of((w + NSLOT * NW) * CHUNK, 8)
        pltpu.sync_copy(edge_hbm.at[0, pl.ds(off, CHUNK)], src0_v)
        pltpu.sync_copy(edge_hbm.at[1, pl.ds(off, CHUNK)], dst0_v)
        pltpu.async_copy(table_hbm.at[src0_v], rows0_v, gsem0).wait()
        pltpu.sync_copy(rows0_v, acc_sh.at[dst0_v], add=True)

    plsc.subcore_barrier()

    def _dump(k, carry):
        bid = s + NS * k

        @pl.when(bid < NB)
        def _():
            off = pl.multiple_of(bid * RB, 8)
            pltpu.sync_copy(acc_sh.at[pl.ds(off, RB)],
                            out_hbm.at[c, pl.ds(off, RB)])

        return carry

    lax.fori_loop(0, (NB + NS - 1) // NS, _dump, 0)


def _sc_pass(table, edge_index):
    mesh = plsc.VectorSubcoreMesh(core_axis_name="c", subcore_axis_name="s")
    kern = pl.kernel(
        _sc_agg_body,
        mesh=mesh,
        out_type=jax.ShapeDtypeStruct((NC, N, D), jnp.float32),
        scratch_types=(
            [pltpu.VMEM((CHUNK,), jnp.int32)] * 12
            + [pltpu.VMEM((CHUNK, D), jnp.float32)] * 3
            + [pltpu.VMEM_SHARED((N, D), jnp.float32)]
            + [pltpu.SemaphoreType.DMA] * 12
        ),
    )
    return kern(table, edge_index)


ROWS_BLK = 1000
GRID = N // ROWS_BLK


def _mlp1_body(pa_ref, w1_ref, b1_ref, w2_ref, b2_ref, out_ref):
    h = pa_ref[0] + pa_ref[1]
    t = jnp.maximum(
        jnp.dot(h, w1_ref[...], preferred_element_type=jnp.float32)
        + b1_ref[...], 0.0)
    o = (jnp.dot(t, w2_ref[...], preferred_element_type=jnp.float32)
         + b2_ref[...])
    out_ref[...] = jnp.maximum(o, 0.0)


def _mlp1(pa, w1, b1, w2, b2):
    return pl.pallas_call(
        _mlp1_body,
        grid=(GRID,),
        in_specs=[
            pl.BlockSpec((NC, ROWS_BLK, D), lambda i: (0, i, 0)),
            pl.BlockSpec((D, D), lambda i: (0, 0)),
            pl.BlockSpec((1, D), lambda i: (0, 0)),
            pl.BlockSpec((D, D), lambda i: (0, 0)),
            pl.BlockSpec((1, D), lambda i: (0, 0)),
        ],
        out_specs=pl.BlockSpec((ROWS_BLK, D), lambda i: (i, 0)),
        out_shape=jax.ShapeDtypeStruct((N, D), jnp.float32),
    )(pa, w1, b1, w2, b2)


def _mlp2_body(pa_ref, w1_ref, b1_ref, w2_ref, b2_ref,
               e_ref, g_ref, out_ref):
    i = pl.program_id(0)
    hin = pa_ref[0] + pa_ref[1]
    t = jnp.maximum(
        jnp.dot(hin, w1_ref[...], preferred_element_type=jnp.float32)
        + b1_ref[...], 0.0)
    h2 = (jnp.dot(t, w2_ref[...], preferred_element_type=jnp.float32)
          + b2_ref[...])
    contrib = lax.dot_general(e_ref[...], h2, (((0,), (0,)), ((), ())),
                              preferred_element_type=jnp.float32)

    @pl.when(i == 0)
    def _():
        out_ref[...] = jnp.zeros_like(out_ref)

    out_ref[...] += contrib

    @pl.when(i == pl.num_programs(0) - 1)
    def _():
        den = jnp.sum(g_ref[...], axis=1, keepdims=True) + 1e-4
        out_ref[...] = out_ref[...] / den


def _mlp2(pa, w1, b1, w2, b2, e_rep, ghg):
    return pl.pallas_call(
        _mlp2_body,
        grid=(GRID,),
        in_specs=[
            pl.BlockSpec((NC, ROWS_BLK, D), lambda i: (0, i, 0)),
            pl.BlockSpec((D, D), lambda i: (0, 0)),
            pl.BlockSpec((1, D), lambda i: (0, 0)),
            pl.BlockSpec((D, D), lambda i: (0, 0)),
            pl.BlockSpec((1, D), lambda i: (0, 0)),
            pl.BlockSpec((ROWS_BLK, G), lambda i: (i, 0)),
            pl.BlockSpec((G, N // GSZ), lambda i: (0, 0)),
        ],
        out_specs=pl.BlockSpec((G, D), lambda i: (0, 0)),
        out_shape=jax.ShapeDtypeStruct((G, D), jnp.float32),
    )(pa, w1, b1, w2, b2, e_rep, ghg)


def kernel(x, edge_index, graph_has_graphlet,
           W1a, b1a, W2a, b2a, W1b, b1b, W2b, b2b):
    b1a2 = b1a.reshape(1, D)
    b2a2 = b2a.reshape(1, D)
    b1b2 = b1b.reshape(1, D)
    b2b2 = b2b.reshape(1, D)
    ghg_rep_t = jnp.repeat(graph_has_graphlet.T, GSZ, axis=0)

    pa1 = _sc_pass(x, edge_index)
    h1r = _mlp1(pa1, W1a, b1a2, W2a, b2a2)
    pa2 = _sc_pass(h1r, edge_index)
    out = _mlp2(pa2, W1b, b1b2, W2b, b2b2, ghg_rep_t, graph_has_graphlet)
    return out
